# baseline (speedup 1.0000x reference)
import functools

import jax
import jax.numpy as jnp
from jax import lax
from jax.experimental import pallas as pl
from jax.experimental.pallas import tpu as pltpu
from jax.experimental.pallas import tpu_sc as plsc

D = 64
K = 10
L = 16
NC = 2
NS = 16
NW = NC * NS
CB = 32
PBW = 8192
NBLK = 32
QS = NBLK * PBW


def _pack_table(table_t):
    d, _ = table_t.shape
    nin = pl.cdiv(table_t.shape[1], PBW)

    def body(a_ref, b_ref, c_ref, e_ref, out_ref):
        def quarter(ref):
            z = lax.bitcast_convert_type(ref[...], jnp.uint32)
            w = (((z[0:32, :] + 0x8000) >> 16) |
                 ((z[32:64, :] + 0x8000) & jnp.uint32(0xFFFF0000)))
            return lax.bitcast_convert_type(w, jnp.float32).T

        out_ref[...] = jnp.concatenate(
            [quarter(a_ref), quarter(b_ref), quarter(c_ref), quarter(e_ref)],
            axis=1)

    def spec(q):
        return pl.BlockSpec(
            (d, PBW), lambda i: (0, jnp.minimum(i + q * NBLK, nin - 1)))

    return pl.pallas_call(
        body,
        grid=(NBLK,),
        in_specs=[spec(0), spec(1), spec(2), spec(3)],
        out_specs=pl.BlockSpec((PBW, 2 * d), lambda i: (i, 0)),
        out_shape=jax.ShapeDtypeStruct((QS, 2 * d), jnp.float32),
    )(table_t, table_t, table_t, table_t)


def _sc_scores(target_idx, context_idx, negative_idx_t, target_rows,
               context_rows):
    B = target_idx.shape[0]
    bpw = B // NW
    nchunks = bpw // CB

    mesh = plsc.VectorSubcoreMesh(
        core_axis_name="c", subcore_axis_name="s", num_cores=NC,
        num_subcores=NS)

    @functools.partial(
        pl.kernel,
        out_type=(
            jax.ShapeDtypeStruct((B * L // 128, 128), jnp.float32),
            jax.ShapeDtypeStruct((NW, bpw * K * L // 128, 128), jnp.float32),
        ),
        mesh=mesh,
        scratch_types=[
            pltpu.VMEM((bpw,), jnp.int32),
            pltpu.VMEM((bpw,), jnp.int32),
            pltpu.VMEM((K * bpw,), jnp.int32),
            pltpu.VMEM((2, CB, D // 2), jnp.float32),
            pltpu.VMEM((2, CB, D // 2), jnp.float32),
            pltpu.VMEM((2, K * CB, D // 2), jnp.float32),
            pltpu.VMEM((2, CB * L // 128, 128), jnp.float32),
            pltpu.VMEM((2, K * CB * L // 128, 128), jnp.float32),
            pltpu.SemaphoreType.DMA,
            pltpu.SemaphoreType.DMA,
            pltpu.SemaphoreType.DMA,
        ],
        compiler_params=pltpu.CompilerParams(
            needs_layout_passes=False, use_tc_tiling_on_sc=False),
    )
    def sc_kernel(tidx_hbm, cidx_hbm, nidx_hbm, temb_hbm, cemb_hbm,
                  pos_hbm, neg_hbm,
                  tiv, civ, niv, trows, crows, nrows, posw, negw,
                  sem0, sem1, semd):
        wid = lax.axis_index("s") * NC + lax.axis_index("c")
        base = wid * bpw
        sems = (sem0, sem1)

        pltpu.sync_copy(tidx_hbm.at[pl.ds(base, bpw)], tiv)
        pltpu.sync_copy(cidx_hbm.at[pl.ds(base, bpw)], civ)
        pltpu.sync_copy(nidx_hbm.at[wid], niv)

        def fire(g):
            b = g % 2
            cb0 = g * CB
            return [
                pltpu.async_copy(temb_hbm.at[tiv.at[pl.ds(cb0, CB)]],
                                 trows.at[b], sems[b]),
                pltpu.async_copy(cemb_hbm.at[civ.at[pl.ds(cb0, CB)]],
                                 crows.at[b], sems[b]),
                pltpu.async_copy(
                    cemb_hbm.at[niv.at[pl.ds(g * K * CB, K * CB)]],
                    nrows.at[b], sems[b]),
            ]

        prow = CB * L // 128
        nrow = K * CB * L // 128

        inflight = {0: fire(0)}
        dumps = {}
        for g in range(nchunks):
            if g + 1 < nchunks:
                inflight[g + 1] = fire(g + 1)
            for cp in inflight.pop(g):
                cp.wait()
            if g - 2 in dumps:
                for cp in dumps.pop(g - 2):
                    cp.wait()
            b = g % 2

            def unpack2(w):
                u = lax.bitcast_convert_type(w, jnp.uint32)
                return (lax.bitcast_convert_type(u << 16, jnp.float32),
                        lax.bitcast_convert_type(u & jnp.uint32(0xFFFF0000),
                                                 jnp.float32))

            def body(i, carry, b=b):
                r = i // 8
                o = (i % 8) * L
                t = [v for j in range(2)
                     for v in unpack2(trows[b, i, pl.ds(j * L, L)])]
                cv = [v for j in range(2)
                      for v in unpack2(crows[b, i, pl.ds(j * L, L)])]
                p = t[0] * cv[0] + t[1] * cv[1] + t[2] * cv[2] + t[3] * cv[3]
                posw[b, r, pl.ds(o, L)] = p
                for k in range(K):
                    n = [v for j in range(2)
                         for v in unpack2(nrows[b, k * CB + i,
                                                pl.ds(j * L, L)])]
                    q = n[0] * t[0] + n[1] * t[1] + n[2] * t[2] + n[3] * t[3]
                    negw[b, k * (CB // 8) + r, pl.ds(o, L)] = q
                return carry

            lax.fori_loop(0, CB, body, 0)

            dumps[g] = [
                pltpu.async_copy(
                    posw.at[b],
                    pos_hbm.at[pl.ds((base + g * CB) * L // 128, prow)],
                    semd),
                pltpu.async_copy(
                    negw.at[b], neg_hbm.at[wid, pl.ds(g * nrow, nrow)], semd),
            ]

        for g in sorted(dumps):
            for cp in dumps[g]:
                cp.wait()

    return sc_kernel(target_idx, context_idx, negative_idx_t,
                     target_rows, context_rows)


def _loss_tc(pos_wide, neg_wide):
    pos2 = pos_wide.reshape(-1, 128)
    neg2 = neg_wide.reshape(-1, 128)
    npos = pos2.shape[0] * 8
    nneg = neg2.shape[0] * 8

    def body(pos_ref, neg_ref, out_ref):
        sel = (lax.broadcasted_iota(jnp.int32, (128, 8), 0) // L ==
               lax.broadcasted_iota(jnp.int32, (128, 8), 1)
               ).astype(jnp.float32)
        p = jnp.dot(pos_ref[...], sel, preferred_element_type=jnp.float32)
        n = jnp.dot(neg_ref[...], sel, preferred_element_type=jnp.float32)
        ls_p = jnp.minimum(p, 0.0) - jnp.log1p(jnp.exp(-jnp.abs(p)))
        ls_n = jnp.minimum(-n, 0.0) - jnp.log1p(jnp.exp(-jnp.abs(n)))
        out_ref[0, 0] = -(jnp.sum(ls_p) / npos + jnp.sum(ls_n) / nneg)

    out = pl.pallas_call(
        body,
        out_shape=jax.ShapeDtypeStruct((1, 1), jnp.float32),
        out_specs=pl.BlockSpec(memory_space=pltpu.SMEM),
    )(pos2, neg2)
    return out[0, 0]


def _remap(idx):
    return ((idx & (QS - 1)) << 2) | (idx >> 18)


def kernel(target_idx, context_idx, negative_idx, target_embeddings,
           context_embeddings):
    target_rows = _pack_table(target_embeddings.T).reshape(4 * QS, D // 2)
    context_rows = _pack_table(context_embeddings.T).reshape(4 * QS, D // 2)
    B = target_idx.shape[0]
    bpw = B // NW
    nidx = (_remap(negative_idx)
            .reshape(NW, bpw // CB, CB, K)
            .transpose(0, 1, 3, 2)
            .reshape(NW, K * bpw))
    pos_scores, neg_scores = _sc_scores(
        _remap(target_idx), _remap(context_idx), nidx,
        target_rows, context_rows)
    return _loss_tc(pos_scores, neg_scores)


# speedup vs baseline: 1.4228x; 1.4228x over previous
import functools

import jax
import jax.numpy as jnp
from jax import lax
from jax.experimental import pallas as pl
from jax.experimental.pallas import tpu as pltpu
from jax.experimental.pallas import tpu_sc as plsc

D = 64
K = 10
L = 16
NC = 2
NS = 16
NW = NC * NS
CB = 32
PBW = 16384
NBLK = 31
HALF = NBLK * PBW


def _pack_table(table_t):
    d, _ = table_t.shape

    nin = pl.cdiv(table_t.shape[1], PBW)

    def body(a_ref, b_ref, out_ref):
        out_ref[...] = jnp.concatenate([a_ref[...], b_ref[...]], axis=0).T

    return pl.pallas_call(
        body,
        grid=(NBLK,),
        in_specs=[
            pl.BlockSpec((d, PBW), lambda i: (0, i)),
            pl.BlockSpec((d, PBW), lambda i: (0, jnp.minimum(i + NBLK,
                                                             nin - 1))),
        ],
        out_specs=pl.BlockSpec((PBW, 2 * d), lambda i: (i, 0)),
        out_shape=jax.ShapeDtypeStruct((HALF, 2 * d), jnp.float32),
    )(table_t, table_t)


def _sc_scores(target_idx, context_idx, negative_idx_t, target_rows,
               context_rows):
    B = target_idx.shape[0]
    bpw = B // NW
    nchunks = bpw // CB

    mesh = plsc.VectorSubcoreMesh(
        core_axis_name="c", subcore_axis_name="s", num_cores=NC,
        num_subcores=NS)

    @functools.partial(
        pl.kernel,
        out_type=(
            jax.ShapeDtypeStruct((B * L // 128, 128), jnp.float32),
            jax.ShapeDtypeStruct((NW, bpw * K * L // 128, 128), jnp.float32),
        ),
        mesh=mesh,
        scratch_types=[
            pltpu.VMEM((bpw,), jnp.int32),
            pltpu.VMEM((bpw,), jnp.int32),
            pltpu.VMEM((K * bpw,), jnp.int32),
            pltpu.VMEM((2, CB, D), jnp.float32),
            pltpu.VMEM((2, CB, D), jnp.float32),
            pltpu.VMEM((2, K * CB, D), jnp.float32),
            pltpu.VMEM((2, CB * L // 128, 128), jnp.float32),
            pltpu.VMEM((2, K * CB * L // 128, 128), jnp.float32),
            pltpu.SemaphoreType.DMA,
            pltpu.SemaphoreType.DMA,
            pltpu.SemaphoreType.DMA,
        ],
        compiler_params=pltpu.CompilerParams(
            needs_layout_passes=False, use_tc_tiling_on_sc=False),
    )
    def sc_kernel(tidx_hbm, cidx_hbm, nidx_hbm, temb_hbm, cemb_hbm,
                  pos_hbm, neg_hbm,
                  tiv, civ, niv, trows, crows, nrows, posw, negw,
                  sem0, sem1, semd):
        wid = lax.axis_index("s") * NC + lax.axis_index("c")
        base = wid * bpw
        sems = (sem0, sem1)

        pltpu.sync_copy(tidx_hbm.at[pl.ds(base, bpw)], tiv)
        pltpu.sync_copy(cidx_hbm.at[pl.ds(base, bpw)], civ)
        pltpu.sync_copy(nidx_hbm.at[wid], niv)

        def fire(g):
            b = g % 2
            cb0 = g * CB
            return [
                pltpu.async_copy(temb_hbm.at[tiv.at[pl.ds(cb0, CB)]],
                                 trows.at[b], sems[b]),
                pltpu.async_copy(cemb_hbm.at[civ.at[pl.ds(cb0, CB)]],
                                 crows.at[b], sems[b]),
                pltpu.async_copy(
                    cemb_hbm.at[niv.at[pl.ds(g * K * CB, K * CB)]],
                    nrows.at[b], sems[b]),
            ]

        prow = CB * L // 128
        nrow = K * CB * L // 128

        inflight = {0: fire(0)}
        dumps = {}
        for g in range(nchunks):
            if g + 1 < nchunks:
                inflight[g + 1] = fire(g + 1)
            for cp in inflight.pop(g):
                cp.wait()
            if g - 2 in dumps:
                for cp in dumps.pop(g - 2):
                    cp.wait()
            b = g % 2

            def body(i, carry, b=b):
                r = i // 8
                o = (i % 8) * L
                t = [trows[b, i, pl.ds(j * L, L)] for j in range(D // L)]
                cv = [crows[b, i, pl.ds(j * L, L)] for j in range(D // L)]
                p = t[0] * cv[0] + t[1] * cv[1] + t[2] * cv[2] + t[3] * cv[3]
                posw[b, r, pl.ds(o, L)] = p
                for k in range(K):
                    n = [nrows[b, k * CB + i, pl.ds(j * L, L)]
                         for j in range(D // L)]
                    q = n[0] * t[0] + n[1] * t[1] + n[2] * t[2] + n[3] * t[3]
                    negw[b, k * (CB // 8) + r, pl.ds(o, L)] = q
                return carry

            lax.fori_loop(0, CB, body, 0)

            dumps[g] = [
                pltpu.async_copy(
                    posw.at[b],
                    pos_hbm.at[pl.ds((base + g * CB) * L // 128, prow)],
                    semd),
                pltpu.async_copy(
                    negw.at[b], neg_hbm.at[wid, pl.ds(g * nrow, nrow)], semd),
            ]

        for g in sorted(dumps):
            for cp in dumps[g]:
                cp.wait()

    return sc_kernel(target_idx, context_idx, negative_idx_t,
                     target_rows, context_rows)


def _loss_tc(pos_wide, neg_wide):
    pos2 = pos_wide.reshape(-1, 128)
    neg2 = neg_wide.reshape(-1, 128)
    npos = pos2.shape[0] * 8
    nneg = neg2.shape[0] * 8

    def body(pos_ref, neg_ref, out_ref):
        sel = (lax.broadcasted_iota(jnp.int32, (128, 8), 0) // L ==
               lax.broadcasted_iota(jnp.int32, (128, 8), 1)
               ).astype(jnp.float32)
        p = jnp.dot(pos_ref[...], sel, preferred_element_type=jnp.float32)
        n = jnp.dot(neg_ref[...], sel, preferred_element_type=jnp.float32)
        ls_p = jnp.minimum(p, 0.0) - jnp.log1p(jnp.exp(-jnp.abs(p)))
        ls_n = jnp.minimum(-n, 0.0) - jnp.log1p(jnp.exp(-jnp.abs(n)))
        out_ref[0, 0] = -(jnp.sum(ls_p) / npos + jnp.sum(ls_n) / nneg)

    out = pl.pallas_call(
        body,
        out_shape=jax.ShapeDtypeStruct((1, 1), jnp.float32),
        out_specs=pl.BlockSpec(memory_space=pltpu.SMEM),
    )(pos2, neg2)
    return out[0, 0]


def _remap(idx):
    return jnp.where(idx < HALF, 2 * idx, 2 * (idx - HALF) + 1)


def kernel(target_idx, context_idx, negative_idx, target_embeddings,
           context_embeddings):
    target_rows = _pack_table(target_embeddings.T).reshape(2 * HALF, D)
    context_rows = _pack_table(context_embeddings.T).reshape(2 * HALF, D)
    B = target_idx.shape[0]
    bpw = B // NW
    nidx = (_remap(negative_idx)
            .reshape(NW, bpw // CB, CB, K)
            .transpose(0, 1, 3, 2)
            .reshape(NW, K * bpw))
    pos_scores, neg_scores = _sc_scores(
        _remap(target_idx), _remap(context_idx), nidx,
        target_rows, context_rows)
    return _loss_tc(pos_scores, neg_scores)
